# baseline (speedup 1.0000x reference)
import jax
import jax.numpy as jnp
from jax import lax
from jax.experimental import pallas as pl
from jax.experimental.pallas import tpu as pltpu
from jax.experimental.pallas import tpu_sc as plsc

EMB_DIM = 64
NUM_CORES = 2
NUM_SUBCORES = 16
NUM_WORKERS = NUM_CORES * NUM_SUBCORES

SEQ = 200
GATHER = 100
K = SEQ // GATHER


def _fire_row(table_hbm, idx_v, rows_buf, gsem, g):
    for j in range(K):
        pltpu.async_copy(
            table_hbm.at[idx_v.at[g * K + j]],
            rows_buf.at[pl.ds(j * GATHER, GATHER)],
            gsem,
        )


def _wait_bytes(table_hbm, buf, sem):
    pltpu.make_async_copy(table_hbm.at[pl.ds(0, buf.shape[0])], buf, sem).wait()


def _emb_body(table_hbm, idx_hbm, out_hbm,
              idx_v, rows0, rows1, gsem0, gsem1, osem0, osem1):
    wid = lax.axis_index("s") * NUM_CORES + lax.axis_index("c")
    nb = out_hbm.shape[0] // NUM_WORKERS
    idx_rows = nb * K
    b0 = wid * nb

    pltpu.sync_copy(idx_hbm.at[pl.ds(wid * idx_rows, idx_rows)], idx_v)

    _fire_row(table_hbm, idx_v, rows0, gsem0, 0)
    _fire_row(table_hbm, idx_v, rows1, gsem1, 1)

    def body(i, carry):
        g = 2 * i
        _wait_bytes(table_hbm, rows0, gsem0)
        pltpu.async_copy(rows0, out_hbm.at[b0 + g], osem0)
        _wait_bytes(table_hbm, rows1, gsem1)
        pltpu.async_copy(rows1, out_hbm.at[b0 + g + 1], osem1)
        _wait_bytes(table_hbm, rows0, osem0)
        _fire_row(table_hbm, idx_v, rows0, gsem0, g + 2)
        _wait_bytes(table_hbm, rows1, osem1)
        _fire_row(table_hbm, idx_v, rows1, gsem1, g + 3)
        return carry

    lax.fori_loop(0, nb // 2 - 1, body, 0)

    g = nb - 2
    _wait_bytes(table_hbm, rows0, gsem0)
    pltpu.async_copy(rows0, out_hbm.at[b0 + g], osem0)
    _wait_bytes(table_hbm, rows1, gsem1)
    pltpu.async_copy(rows1, out_hbm.at[b0 + g + 1], osem1)
    _wait_bytes(table_hbm, rows0, osem0)
    _wait_bytes(table_hbm, rows1, osem1)


@jax.jit
def _embedding_lookup(word_embeddings, idx2d):
    n_idx_rows = idx2d.shape[0]
    batch = n_idx_rows // K
    idx_rows_per_w = n_idx_rows // NUM_WORKERS

    mesh = plsc.VectorSubcoreMesh(
        core_axis_name="c", subcore_axis_name="s",
        num_cores=NUM_CORES, num_subcores=NUM_SUBCORES)

    grid_kernel = pl.kernel(
        _emb_body,
        out_type=jax.ShapeDtypeStruct((batch, SEQ, EMB_DIM), jnp.float32),
        mesh=mesh,
        compiler_params=pltpu.CompilerParams(use_tc_tiling_on_sc=False),
        scratch_types=[
            pltpu.VMEM((idx_rows_per_w, GATHER), jnp.int32),
            pltpu.VMEM((SEQ, EMB_DIM), jnp.float32),
            pltpu.VMEM((SEQ, EMB_DIM), jnp.float32),
            pltpu.SemaphoreType.DMA,
            pltpu.SemaphoreType.DMA,
            pltpu.SemaphoreType.DMA,
            pltpu.SemaphoreType.DMA,
        ],
    )
    return grid_kernel(word_embeddings, idx2d)


def kernel(input_ids, word_embeddings):
    batch, seq = input_ids.shape
    idx2d = input_ids.reshape(batch * K, GATHER)
    return _embedding_lookup(word_embeddings, idx2d)


# speedup vs baseline: 1.0125x; 1.0125x over previous
import functools

import jax
import jax.numpy as jnp
from jax import lax
from jax.experimental import pallas as pl
from jax.experimental.pallas import tpu as pltpu
from jax.experimental.pallas import tpu_sc as plsc

EMB_DIM = 64
NUM_CORES = 2
NUM_SUBCORES = 16
NUM_WORKERS = NUM_CORES * NUM_SUBCORES

CHUNK = 512
GATHER = 128
K = CHUNK // GATHER


def _fire_chunk(table_hbm, idx_v, rows_buf, gsem, g):
    for j in range(K):
        pltpu.async_copy(
            table_hbm.at[idx_v.at[g * K + j]],
            rows_buf.at[pl.ds(j * GATHER, GATHER)],
            gsem,
        )


def _wait_bytes(table_hbm, buf, sem):
    pltpu.make_async_copy(table_hbm.at[pl.ds(0, buf.shape[0])], buf, sem).wait()


def _emb_body(nchunks, table_hbm, idx_hbm, out_hbm,
              idx_v, rows0, rows1, gsem0, gsem1, osem0, osem1):
    wid = lax.axis_index("s") * NUM_CORES + lax.axis_index("c")
    rows_per_w = nchunks * CHUNK
    idx_rows = rows_per_w // GATHER
    base = wid * rows_per_w

    pltpu.sync_copy(idx_hbm.at[pl.ds(wid * idx_rows, idx_rows)], idx_v)

    def out_start(g):
        return base + g * CHUNK

    _fire_chunk(table_hbm, idx_v, rows0, gsem0, 0)
    _fire_chunk(table_hbm, idx_v, rows1, gsem1, 1)

    def body(i, carry):
        g = 2 * i
        _wait_bytes(table_hbm, rows0, gsem0)
        pltpu.async_copy(rows0, out_hbm.at[pl.ds(out_start(g), CHUNK)], osem0)
        _wait_bytes(table_hbm, rows1, gsem1)
        pltpu.async_copy(rows1, out_hbm.at[pl.ds(out_start(g + 1), CHUNK)],
                         osem1)
        _wait_bytes(table_hbm, rows0, osem0)
        _fire_chunk(table_hbm, idx_v, rows0, gsem0, g + 2)
        _wait_bytes(table_hbm, rows1, osem1)
        _fire_chunk(table_hbm, idx_v, rows1, gsem1, g + 3)
        return carry

    lax.fori_loop(0, nchunks // 2 - 1, body, 0)

    g = nchunks - 2
    _wait_bytes(table_hbm, rows0, gsem0)
    pltpu.async_copy(rows0, out_hbm.at[pl.ds(out_start(g), CHUNK)], osem0)
    _wait_bytes(table_hbm, rows1, gsem1)
    pltpu.async_copy(rows1, out_hbm.at[pl.ds(out_start(g + 1), CHUNK)], osem1)
    _wait_bytes(table_hbm, rows0, osem0)
    _wait_bytes(table_hbm, rows1, osem1)


@jax.jit
def _embedding_lookup(word_embeddings, idx2d):
    n_idx_rows = idx2d.shape[0]
    total_rows = n_idx_rows * GATHER
    nchunks = total_rows // (NUM_WORKERS * CHUNK)
    idx_rows_per_w = n_idx_rows // NUM_WORKERS

    mesh = plsc.VectorSubcoreMesh(
        core_axis_name="c", subcore_axis_name="s",
        num_cores=NUM_CORES, num_subcores=NUM_SUBCORES)

    grid_kernel = pl.kernel(
        functools.partial(_emb_body, nchunks),
        out_type=jax.ShapeDtypeStruct((total_rows, EMB_DIM), jnp.float32),
        mesh=mesh,
        compiler_params=pltpu.CompilerParams(use_tc_tiling_on_sc=False),
        scratch_types=[
            pltpu.VMEM((idx_rows_per_w, GATHER), jnp.int32),
            pltpu.VMEM((CHUNK, EMB_DIM), jnp.float32),
            pltpu.VMEM((CHUNK, EMB_DIM), jnp.float32),
            pltpu.SemaphoreType.DMA,
            pltpu.SemaphoreType.DMA,
            pltpu.SemaphoreType.DMA,
            pltpu.SemaphoreType.DMA,
        ],
    )
    return grid_kernel(word_embeddings, idx2d)


def kernel(input_ids, word_embeddings):
    batch, seq = input_ids.shape
    total = batch * seq
    idx2d = input_ids.reshape(total // GATHER, GATHER)
    out = _embedding_lookup(word_embeddings, idx2d)
    return out.reshape(batch, seq, EMB_DIM)


# speedup vs baseline: 1.0130x; 1.0005x over previous
import functools

import jax
import jax.numpy as jnp
from jax import lax
from jax.experimental import pallas as pl
from jax.experimental.pallas import tpu as pltpu
from jax.experimental.pallas import tpu_sc as plsc

EMB_DIM = 64
NUM_CORES = 2
NUM_SUBCORES = 16
NUM_WORKERS = NUM_CORES * NUM_SUBCORES

CHUNK = 640
GATHER = 128
K = CHUNK // GATHER


def _fire_chunk(table_hbm, idx_v, rows_buf, gsem, g):
    for j in range(K):
        pltpu.async_copy(
            table_hbm.at[idx_v.at[g * K + j]],
            rows_buf.at[pl.ds(j * GATHER, GATHER)],
            gsem,
        )


def _wait_bytes(table_hbm, buf, sem):
    pltpu.make_async_copy(table_hbm.at[pl.ds(0, buf.shape[0])], buf, sem).wait()


def _emb_body(nchunks, table_hbm, idx_hbm, out_hbm,
              idx_v, rows0, rows1, gsem0, gsem1, osem0, osem1):
    wid = lax.axis_index("s") * NUM_CORES + lax.axis_index("c")
    rows_per_w = nchunks * CHUNK
    idx_rows = rows_per_w // GATHER
    base = wid * rows_per_w

    pltpu.sync_copy(idx_hbm.at[pl.ds(wid * idx_rows, idx_rows)], idx_v)

    def out_start(g):
        return base + g * CHUNK

    _fire_chunk(table_hbm, idx_v, rows0, gsem0, 0)
    _fire_chunk(table_hbm, idx_v, rows1, gsem1, 1)

    def body(i, carry):
        g = 2 * i
        _wait_bytes(table_hbm, rows0, gsem0)
        pltpu.async_copy(rows0, out_hbm.at[pl.ds(out_start(g), CHUNK)], osem0)
        _wait_bytes(table_hbm, rows1, gsem1)
        pltpu.async_copy(rows1, out_hbm.at[pl.ds(out_start(g + 1), CHUNK)],
                         osem1)
        _wait_bytes(table_hbm, rows0, osem0)
        _fire_chunk(table_hbm, idx_v, rows0, gsem0, g + 2)
        _wait_bytes(table_hbm, rows1, osem1)
        _fire_chunk(table_hbm, idx_v, rows1, gsem1, g + 3)
        return carry

    lax.fori_loop(0, nchunks // 2 - 1, body, 0)

    g = nchunks - 2
    _wait_bytes(table_hbm, rows0, gsem0)
    pltpu.async_copy(rows0, out_hbm.at[pl.ds(out_start(g), CHUNK)], osem0)
    _wait_bytes(table_hbm, rows1, gsem1)
    pltpu.async_copy(rows1, out_hbm.at[pl.ds(out_start(g + 1), CHUNK)], osem1)
    _wait_bytes(table_hbm, rows0, osem0)
    _wait_bytes(table_hbm, rows1, osem1)


@jax.jit
def _embedding_lookup(word_embeddings, idx2d):
    n_idx_rows = idx2d.shape[0]
    total_rows = n_idx_rows * GATHER
    nchunks = total_rows // (NUM_WORKERS * CHUNK)
    idx_rows_per_w = n_idx_rows // NUM_WORKERS

    mesh = plsc.VectorSubcoreMesh(
        core_axis_name="c", subcore_axis_name="s",
        num_cores=NUM_CORES, num_subcores=NUM_SUBCORES)

    grid_kernel = pl.kernel(
        functools.partial(_emb_body, nchunks),
        out_type=jax.ShapeDtypeStruct((total_rows, EMB_DIM), jnp.float32),
        mesh=mesh,
        compiler_params=pltpu.CompilerParams(use_tc_tiling_on_sc=False),
        scratch_types=[
            pltpu.VMEM((idx_rows_per_w, GATHER), jnp.int32),
            pltpu.VMEM((CHUNK, EMB_DIM), jnp.float32),
            pltpu.VMEM((CHUNK, EMB_DIM), jnp.float32),
            pltpu.SemaphoreType.DMA,
            pltpu.SemaphoreType.DMA,
            pltpu.SemaphoreType.DMA,
            pltpu.SemaphoreType.DMA,
        ],
    )
    return grid_kernel(word_embeddings, idx2d)


def kernel(input_ids, word_embeddings):
    batch, seq = input_ids.shape
    total = batch * seq
    idx2d = input_ids.reshape(total // GATHER, GATHER)
    out = _embedding_lookup(word_embeddings, idx2d)
    return out.reshape(batch, seq, EMB_DIM)
